# Initial kernel scaffold; baseline (speedup 1.0000x reference)
import jax
import jax.numpy as jnp
from jax.experimental import pallas as pl


def kernel(x, table):
    raise NotImplementedError("write your pallas kernel here")


# speedup vs baseline: 1.2153x; 1.2153x over previous
import functools

import jax
import jax.numpy as jnp
from jax import lax
from jax.experimental import pallas as pl
from jax.experimental.pallas import tpu as pltpu
from jax.experimental.pallas import tpu_sc as plsc

_NC = 2
_NS = 16
_NW = _NC * _NS
_CH = 128


def kernel(x, table):
    batch, hist = x.shape
    vocab, dim = table.shape
    n = batch * hist
    per_w = n // _NW
    nch = per_w // _CH
    assert per_w * _NW == n and nch * _CH == per_w
    idx = x.reshape(n).astype(jnp.int32)
    mesh = plsc.VectorSubcoreMesh(core_axis_name="c", subcore_axis_name="s")

    @functools.partial(
        pl.kernel,
        mesh=mesh,
        out_type=jax.ShapeDtypeStruct((n, dim), table.dtype),
        compiler_params=pltpu.CompilerParams(use_tc_tiling_on_sc=False),
        scratch_types=[
            pltpu.VMEM((_CH,), jnp.int32),
            pltpu.VMEM((_CH, dim), jnp.float32),
            pltpu.SemaphoreType.DMA,
        ],
    )
    def gather_kernel(table_hbm, idx_hbm, out_hbm, idx_v, rows_v, sem):
        wid = lax.axis_index("s") * _NC + lax.axis_index("c")
        base = wid * per_w

        @pl.loop(0, nch)
        def _(g):
            off = base + g * _CH
            pltpu.sync_copy(idx_hbm.at[pl.ds(off, _CH)], idx_v)
            pltpu.async_copy(table_hbm.at[idx_v], rows_v, sem).wait()
            pltpu.sync_copy(rows_v, out_hbm.at[pl.ds(off, _CH)])

    out = gather_kernel(table, idx)
    return out.reshape(batch, hist, dim)


# speedup vs baseline: 1.4916x; 1.2273x over previous
import functools

import jax
import jax.numpy as jnp
from jax import lax
from jax.experimental import pallas as pl
from jax.experimental.pallas import tpu as pltpu
from jax.experimental.pallas import tpu_sc as plsc

_NC = 2
_NS = 16
_NW = _NC * _NS
_CH = 128
_K = 10


def kernel(x, table):
    batch, hist = x.shape
    vocab, dim = table.shape
    n = batch * hist
    per_w = n // _NW
    nsuper = per_w // (_K * _CH)
    assert per_w * _NW == n and nsuper * _K * _CH == per_w and nsuper % 2 == 0
    idx = x.reshape(n).astype(jnp.int32)
    mesh = plsc.VectorSubcoreMesh(core_axis_name="c", subcore_axis_name="s")

    @functools.partial(
        pl.kernel,
        mesh=mesh,
        out_type=jax.ShapeDtypeStruct((n, dim), table.dtype),
        compiler_params=pltpu.CompilerParams(use_tc_tiling_on_sc=False),
        scratch_types=[
            pltpu.VMEM((per_w,), jnp.int32),
            pltpu.VMEM((2, _K, _CH, dim), jnp.float32),
            pltpu.SemaphoreType.DMA,
            pltpu.SemaphoreType.DMA,
            pltpu.SemaphoreType.DMA,
            pltpu.SemaphoreType.DMA,
        ],
    )
    def gather_kernel(table_hbm, idx_hbm, out_hbm, idx_v, rows_v,
                      gsem0, gsem1, osem0, osem1):
        wid = lax.axis_index("s") * _NC + lax.axis_index("c")
        base = wid * per_w
        pltpu.sync_copy(idx_hbm.at[pl.ds(base, per_w)], idx_v)
        gsems = (gsem0, gsem1)
        osems = (osem0, osem1)

        def fire_gathers(s, b):
            for j in range(_K):
                off = s * (_K * _CH) + j * _CH
                pltpu.async_copy(table_hbm.at[idx_v.at[pl.ds(off, _CH)]],
                                 rows_v.at[b].at[j], gsems[b])

        def drain_gathers(s, b):
            for j in range(_K):
                off = s * (_K * _CH) + j * _CH
                pltpu.make_async_copy(table_hbm.at[idx_v.at[pl.ds(off, _CH)]],
                                      rows_v.at[b].at[j], gsems[b]).wait()

        def fire_stores(s, b):
            for j in range(_K):
                off = base + s * (_K * _CH) + j * _CH
                pltpu.async_copy(rows_v.at[b].at[j],
                                 out_hbm.at[pl.ds(off, _CH)], osems[b])

        def drain_stores(s, b):
            for j in range(_K):
                off = base + s * (_K * _CH) + j * _CH
                pltpu.make_async_copy(rows_v.at[b].at[j],
                                      out_hbm.at[pl.ds(off, _CH)],
                                      osems[b]).wait()

        for s in range(2):
            fire_gathers(s, s)
            drain_gathers(s, s)
            fire_stores(s, s)

        @pl.loop(2, nsuper, step=2)
        def _(s0):
            for b in range(2):
                s = s0 + b
                drain_stores(s - 2, b)
                fire_gathers(s, b)
                drain_gathers(s, b)
                fire_stores(s, b)

        drain_stores(nsuper - 2, 0)
        drain_stores(nsuper - 1, 1)

    out = gather_kernel(table, idx)
    return out.reshape(batch, hist, dim)
